# baseline (speedup 1.0000x reference)
import functools

import jax
import jax.numpy as jnp
import numpy as np
from jax import lax
from jax.experimental import pallas as pl
from jax.experimental.pallas import tpu as pltpu
from jax.experimental.pallas import tpu_sc as plsc

D_MODEL = 1024
SEQ_LEN = 2048
HALF = D_MODEL // 2

_PE_SPECS = (
    (4096, 2048, -1),
    (2048, 2048, -2),
    (4096, 2047, -1),
    (1024, 2048, -4),
    (2048, 2046, -2),
    (512, 2048, -8),
    (1024, 2044, -4),
)

_NC, _NS = 2, 16
_NW = _NC * _NS
_TBL_R = 512
_TBL_STEPS = 9
_TBL_ROWS = _TBL_R * _TBL_STEPS

_NPW = [n // _NW for (n, _, _) in _PE_SPECS]
_SEG = [0]
for _n in _NPW:
    _SEG.append(_SEG[-1] + _n)
_ROWS_PW = _SEG[-1]
_IDX_BASE = [0]
for (_n, _, _) in _PE_SPECS:
    _IDX_BASE.append(_IDX_BASE[-1] + _n)

_CHUNK = 64


def _table_body(vals_ref, invf_ref, ref):
    v = vals_ref[0, 0, :]
    invf = invf_ref[0, :]
    arg8 = v[:8][:, None] * invf[None, :]
    ref[0:8, :HALF] = jnp.sin(arg8)
    ref[0:8, HALF:] = jnp.cos(arg8)
    d = v[1:2] - v[0:1]
    n = 8
    while n < _TBL_R:
        rot = (n * d)[:, None] * invf[None, :]
        rs = jnp.sin(rot)
        rc = jnp.cos(rot)
        s = ref[0:n, :HALF]
        c = ref[0:n, HALF:]
        ref[n:2 * n, :HALF] = s * rc + c * rs
        ref[n:2 * n, HALF:] = c * rc - s * rs
        n *= 2


def _build_table(dtype):
    vals = (SEQ_LEN - np.arange(_TBL_ROWS, dtype=np.float32)).reshape(
        _TBL_STEPS, 1, _TBL_R)
    vals = jnp.asarray(vals, dtype=dtype)
    freq = jnp.arange(HALF, dtype=dtype)
    invf = (1.0 / (10000.0 ** (freq / HALF)))[None, :]
    return pl.pallas_call(
        _table_body,
        grid=(_TBL_STEPS,),
        in_specs=[
            pl.BlockSpec((1, 1, _TBL_R), lambda i: (i, 0, 0)),
            pl.BlockSpec((1, HALF), lambda i: (0, 0)),
        ],
        out_specs=pl.BlockSpec((_TBL_R, D_MODEL), lambda i: (i, 0)),
        out_shape=jax.ShapeDtypeStruct((_TBL_ROWS, D_MODEL), dtype),
    )(vals, invf)


_STRIDE = [-step for (_, _, step) in _PE_SPECS]
_Q0 = [
    (SEQ_LEN - first) // (-step) for (_, first, step) in _PE_SPECS
]


def _sc_body(*refs):
    tviews = {1: refs[0], 2: refs[1], 4: refs[2], 8: refs[3]}
    outs = refs[4:4 + len(_PE_SPECS)]
    sem = refs[4 + len(_PE_SPECS)]
    wid = lax.axis_index("s") * _NC + lax.axis_index("c")
    copies = []
    for k, ref in enumerate(outs):
        npw = _NPW[k]
        woff = wid * npw
        src = tviews[_STRIDE[k]].at[pl.ds(_Q0[k] + woff, npw), pl.ds(0, 1)]
        copies.append(pltpu.async_copy(src, ref.at[pl.ds(woff, npw)], sem))
    for cp in copies:
        cp.wait()


def _sc_gather(table, dtype):
    mesh = plsc.VectorSubcoreMesh(
        core_axis_name="c", subcore_axis_name="s",
        num_cores=_NC, num_subcores=_NS)
    views = [table.reshape(_TBL_ROWS // s, s, D_MODEL) for s in (1, 2, 4, 8)]
    out_type = [
        jax.ShapeDtypeStruct((n, 1, D_MODEL), dtype) for (n, _, _) in _PE_SPECS
    ]
    outs = pl.kernel(
        _sc_body,
        out_type,
        mesh=mesh,
        scratch_types=[pltpu.SemaphoreType.DMA],
    )(*views)
    return [o.reshape(o.shape[0], D_MODEL) for o in outs]


_TT_ROWS = 512


def _tt_body(row_ref, full_ref, ttm_ref, cls_ref):
    j = pl.program_id(0)
    b = pl.program_id(1)
    shape = (_TT_ROWS, SEQ_LEN)
    rows = jnp.broadcast_to(row_ref[0, 0, :][:, None], shape)
    cols = jnp.broadcast_to(full_ref[0, 0, :][None, :], shape)
    ttm_ref[0] = (rows == cols) | (rows == 2) | (cols == 2)

    @pl.when(b == 0)
    def _():
        ri = jax.lax.broadcasted_iota(jnp.int32, shape, 0)
        ci = jax.lax.broadcasted_iota(jnp.int32, shape, 1)
        cls_ref[...] = (((ri + j * _TT_ROWS) > 0) & (ci > 0)).astype(cls_ref.dtype)


def _build_ttm(token_type_ids, dtype):
    batch = token_type_ids.shape[0]
    ids3 = token_type_ids.reshape(batch, 1, SEQ_LEN)
    nj = SEQ_LEN // _TT_ROWS
    return pl.pallas_call(
        _tt_body,
        grid=(nj, batch),
        in_specs=[
            pl.BlockSpec((1, 1, _TT_ROWS), lambda j, b: (b, 0, j)),
            pl.BlockSpec((1, 1, SEQ_LEN), lambda j, b: (b, 0, 0)),
        ],
        out_specs=[
            pl.BlockSpec((1, _TT_ROWS, SEQ_LEN), lambda j, b: (b, j, 0)),
            pl.BlockSpec((_TT_ROWS, SEQ_LEN), lambda j, b: (j, 0)),
        ],
        out_shape=[
            jax.ShapeDtypeStruct((batch, SEQ_LEN, SEQ_LEN), jnp.bool_),
            jax.ShapeDtypeStruct((SEQ_LEN, SEQ_LEN), dtype),
        ],
    )(ids3, ids3)


_IDX_ALL = np.concatenate([
    SEQ_LEN - (first + step * np.arange(n))
    for (n, first, step) in _PE_SPECS
]).astype(np.int32)


def kernel(inputs_embeds, attention_mask, token_type_ids):
    dtype = inputs_embeds.dtype
    table = _build_table(dtype)
    pes = _sc_gather(table, dtype)
    ttm, cls_mask = _build_ttm(token_type_ids, dtype)
    return (*pes, ttm, attention_mask, cls_mask)


# speedup vs baseline: 15.3917x; 15.3917x over previous
import functools

import jax
import jax.numpy as jnp
import numpy as np
from jax import lax
from jax.experimental import pallas as pl
from jax.experimental.pallas import tpu as pltpu
from jax.experimental.pallas import tpu_sc as plsc

D_MODEL = 1024
SEQ_LEN = 2048
HALF = D_MODEL // 2

_PE_SPECS = (
    (4096, 2048, -1),
    (2048, 2048, -2),
    (4096, 2047, -1),
    (1024, 2048, -4),
    (2048, 2046, -2),
    (512, 2048, -8),
    (1024, 2044, -4),
)

_NC, _NS = 2, 16
_NW = _NC * _NS
_TBL_R = 512
_TBL_STEPS = 9
_TBL_ROWS = _TBL_R * _TBL_STEPS

_NPW = [n // _NW for (n, _, _) in _PE_SPECS]
_SEG = [0]
for _n in _NPW:
    _SEG.append(_SEG[-1] + _n)
_ROWS_PW = _SEG[-1]
_IDX_BASE = [0]
for (_n, _, _) in _PE_SPECS:
    _IDX_BASE.append(_IDX_BASE[-1] + _n)

_CHUNK = 48


def _table_body(vals_ref, invf_ref, ref):
    v = vals_ref[0, 0, :]
    invf = invf_ref[0, :]
    arg8 = v[:8][:, None] * invf[None, :]
    ref[0:8, :HALF] = jnp.sin(arg8)
    ref[0:8, HALF:] = jnp.cos(arg8)
    d = v[1:2] - v[0:1]
    n = 8
    while n < _TBL_R:
        rot = (n * d)[:, None] * invf[None, :]
        rs = jnp.sin(rot)
        rc = jnp.cos(rot)
        s = ref[0:n, :HALF]
        c = ref[0:n, HALF:]
        ref[n:2 * n, :HALF] = s * rc + c * rs
        ref[n:2 * n, HALF:] = c * rc - s * rs
        n *= 2


def _build_table(dtype):
    vals = (SEQ_LEN - np.arange(_TBL_ROWS, dtype=np.float32)).reshape(
        _TBL_STEPS, 1, _TBL_R)
    vals = jnp.asarray(vals, dtype=dtype)
    freq = jnp.arange(HALF, dtype=dtype)
    invf = (1.0 / (10000.0 ** (freq / HALF)))[None, :]
    return pl.pallas_call(
        _table_body,
        grid=(_TBL_STEPS,),
        in_specs=[
            pl.BlockSpec((1, 1, _TBL_R), lambda i: (i, 0, 0)),
            pl.BlockSpec((1, HALF), lambda i: (0, 0)),
        ],
        out_specs=pl.BlockSpec((_TBL_R, D_MODEL), lambda i: (i, 0)),
        out_shape=jax.ShapeDtypeStruct((_TBL_ROWS, D_MODEL), dtype),
    )(vals, invf)


_CHUNKS = []
for _k in range(len(_PE_SPECS)):
    _co = 0
    while _co < _NPW[_k]:
        _cs = min(_CHUNK, _NPW[_k] - _co)
        _CHUNKS.append((_k, _co, _cs))
        _co += _cs


def _sc_body(table_ref, idx_ref, *rest):
    outs = rest[:len(_PE_SPECS)]
    idx_v, b0, b1, g0, g1, s0, s1 = rest[len(_PE_SPECS):]
    bufs, gsems, ssems = [b0, b1], [g0, g1], [s0, s1]
    wid = lax.axis_index("s") * _NC + lax.axis_index("c")
    for k in range(len(_PE_SPECS)):
        npw = _NPW[k]
        pltpu.sync_copy(
            idx_ref.at[pl.ds(_IDX_BASE[k] + wid * npw, npw)],
            idx_v.at[pl.ds(_SEG[k], npw)],
        )

    def gstart(c):
        k, co, cs = _CHUNKS[c]
        return pltpu.async_copy(
            table_ref.at[idx_v.at[pl.ds(_SEG[k] + co, cs)]],
            bufs[c % 2].at[pl.ds(0, cs)],
            gsems[c % 2],
        )

    def sstart(c):
        k, co, cs = _CHUNKS[c]
        return pltpu.async_copy(
            bufs[c % 2].at[pl.ds(0, cs)],
            outs[k].at[pl.ds(wid * _NPW[k] + co, cs)],
            ssems[c % 2],
        )

    n = len(_CHUNKS)
    gd = [None] * n
    sd = [None] * n
    gd[0] = gstart(0)
    for c in range(n):
        gd[c].wait()
        if c + 1 < n:
            if c >= 1:
                sd[c - 1].wait()
            gd[c + 1] = gstart(c + 1)
        sd[c] = sstart(c)
    if n >= 2:
        sd[n - 2].wait()
    sd[n - 1].wait()


def _sc_gather(table, idx_all, dtype):
    mesh = plsc.VectorSubcoreMesh(
        core_axis_name="c", subcore_axis_name="s",
        num_cores=_NC, num_subcores=_NS)
    out_type = [jax.ShapeDtypeStruct((n, D_MODEL), dtype) for (n, _, _) in _PE_SPECS]
    return pl.kernel(
        _sc_body,
        out_type,
        mesh=mesh,
        scratch_types=[
            pltpu.VMEM((_ROWS_PW,), jnp.int32),
            pltpu.VMEM((_CHUNK, D_MODEL), jnp.float32),
            pltpu.VMEM((_CHUNK, D_MODEL), jnp.float32),
            pltpu.SemaphoreType.DMA,
            pltpu.SemaphoreType.DMA,
            pltpu.SemaphoreType.DMA,
            pltpu.SemaphoreType.DMA,
        ],
    )(table, idx_all)


_TT_ROWS = 512


def _tt_body(row_ref, full_ref, ttm_ref, cls_ref):
    j = pl.program_id(0)
    b = pl.program_id(1)
    shape = (_TT_ROWS, SEQ_LEN)
    rows = jnp.broadcast_to(row_ref[0, 0, :][:, None], shape)
    cols = jnp.broadcast_to(full_ref[0, 0, :][None, :], shape)
    ttm_ref[0] = (rows == cols) | (rows == 2) | (cols == 2)

    @pl.when(b == 0)
    def _():
        ri = jax.lax.broadcasted_iota(jnp.int32, shape, 0)
        ci = jax.lax.broadcasted_iota(jnp.int32, shape, 1)
        cls_ref[...] = (((ri + j * _TT_ROWS) > 0) & (ci > 0)).astype(cls_ref.dtype)


def _build_ttm(token_type_ids, dtype):
    batch = token_type_ids.shape[0]
    ids3 = token_type_ids.reshape(batch, 1, SEQ_LEN)
    nj = SEQ_LEN // _TT_ROWS
    return pl.pallas_call(
        _tt_body,
        grid=(nj, batch),
        in_specs=[
            pl.BlockSpec((1, 1, _TT_ROWS), lambda j, b: (b, 0, j)),
            pl.BlockSpec((1, 1, SEQ_LEN), lambda j, b: (b, 0, 0)),
        ],
        out_specs=[
            pl.BlockSpec((1, _TT_ROWS, SEQ_LEN), lambda j, b: (b, j, 0)),
            pl.BlockSpec((_TT_ROWS, SEQ_LEN), lambda j, b: (j, 0)),
        ],
        out_shape=[
            jax.ShapeDtypeStruct((batch, SEQ_LEN, SEQ_LEN), jnp.bool_),
            jax.ShapeDtypeStruct((SEQ_LEN, SEQ_LEN), dtype),
        ],
    )(ids3, ids3)


_IDX_ALL = np.concatenate([
    SEQ_LEN - (first + step * np.arange(n))
    for (n, first, step) in _PE_SPECS
]).astype(np.int32)


def kernel(inputs_embeds, attention_mask, token_type_ids):
    dtype = inputs_embeds.dtype
    table = _build_table(dtype)
    pes = _sc_gather(table, jnp.asarray(_IDX_ALL), dtype)
    ttm, cls_mask = _build_ttm(token_type_ids, dtype)
    return (*pes, ttm, attention_mask, cls_mask)
